# baseline (speedup 1.0000x reference)
import functools

import jax
import jax.numpy as jnp
from jax import lax
from jax.experimental import pallas as pl
from jax.experimental.pallas import tpu as pltpu
from jax.experimental.pallas import tpu_sc as plsc

_G = 17911
_B, _N, _D = 4, 8192, 768
_EMB = 128
_TOP = 5000
_RTOT = _B * _TOP

def _mono(v):
    b = lax.bitcast_convert_type(v, jnp.int32)
    return jnp.where(b >= 0, b, jnp.full_like(b, 2147483647) - b)


_KB = 1024
_NB = _N // _KB
_GBF = 4480


def _rank_body(xrow_ref, grn_ref, ppi_ref, w_ref, b_ref, o_ref, f_ref,
               mrow, macc):
    w = w_ref[...]
    f_ref[...] = (
        jnp.dot(grn_ref[...], w[:_EMB, :], preferred_element_type=jnp.float32)
        + jnp.dot(ppi_ref[...], w[_EMB:, :], preferred_element_type=jnp.float32)
        + b_ref[...]
    )

    mrow[0, :] = _mono(xrow_ref[0, 0, :])
    ii = lax.broadcasted_iota(jnp.int32, (_KB, _KB), 1)
    jj = lax.broadcasted_iota(jnp.int32, (_KB, _KB), 0)
    tie_mask = jj < ii

    def a_loop(a, carry):
        mi = mrow[0, pl.ds(a * _KB, _KB)][None, :]
        macc[...] = ((mrow[0, pl.ds(a * _KB, _KB)][:, None] > mi)
                     | ((mrow[0, pl.ds(a * _KB, _KB)][:, None] == mi)
                        & tie_mask)).astype(jnp.float32)

        def lo_loop(b, carry2):
            mj = mrow[0, pl.ds(b * _KB, _KB)][:, None]
            macc[...] += (mj >= mi).astype(jnp.float32)
            return carry2

        def hi_loop(b, carry2):
            mj = mrow[0, pl.ds(b * _KB, _KB)][:, None]
            macc[...] += (mj > mi).astype(jnp.float32)
            return carry2

        lax.fori_loop(0, a, lo_loop, 0)
        lax.fori_loop(a + 1, _NB, hi_loop, 0)
        o_ref[0, 0, pl.ds(a * _KB, _KB)] = jnp.sum(
            macc[...], axis=0).astype(jnp.int32)
        return carry

    lax.fori_loop(0, _NB, a_loop, 0)


def _rank_fc1_call(x_raw, grn, ppi, w, b2d):
    x3 = x_raw.reshape(_B, 1, _N)
    ranks, fc1 = pl.pallas_call(
        _rank_body,
        grid=(_B,),
        in_specs=[
            pl.BlockSpec((1, 1, _N), lambda b: (b, 0, 0)),
            pl.BlockSpec((_GBF, _EMB), lambda b: (b, 0)),
            pl.BlockSpec((_GBF, _EMB), lambda b: (b, 0)),
            pl.BlockSpec((2 * _EMB, _EMB), lambda b: (0, 0)),
            pl.BlockSpec((1, _EMB), lambda b: (0, 0)),
        ],
        out_specs=[
            pl.BlockSpec((1, 1, _N), lambda b: (b, 0, 0)),
            pl.BlockSpec((_GBF, _EMB), lambda b: (b, 0)),
        ],
        out_shape=[
            jax.ShapeDtypeStruct((_B, 1, _N), jnp.int32),
            jax.ShapeDtypeStruct((_G, _EMB), jnp.float32),
        ],
        scratch_shapes=[
            pltpu.VMEM((1, _N), jnp.int32),
            pltpu.VMEM((_KB, _KB), jnp.float32),
        ],
    )(x3, grn, ppi, w, b2d)
    return ranks.reshape(_B, _N), fc1


_CH = 2048
_PC = 200
_PCX = 40


def _sc_select_gather(b0, ranks, xraw, xind, xflat, fc1):
    mesh = plsc.VectorSubcoreMesh(core_axis_name="c", subcore_axis_name="s")
    half_rows = 2 * _TOP
    nchunks = _TOP // _PC

    @functools.partial(
        pl.kernel,
        mesh=mesh,
        compiler_params=pltpu.CompilerParams(needs_layout_passes=False),
        out_type=[
            jax.ShapeDtypeStruct((half_rows, _D), jnp.float32),
            jax.ShapeDtypeStruct((half_rows, _EMB), jnp.float32),
            jax.ShapeDtypeStruct((half_rows,), jnp.float32),
            jax.ShapeDtypeStruct((half_rows,), jnp.int32),
            jax.ShapeDtypeStruct((half_rows,), jnp.int32),
        ],
        scratch_types=[
            pltpu.VMEM((_CH,), jnp.int32),
            pltpu.VMEM((_CH,), jnp.float32),
            pltpu.VMEM((_CH,), jnp.int32),
            pltpu.VMEM((_TOP,), jnp.int32),
            pltpu.VMEM((_TOP,), jnp.float32),
            pltpu.VMEM((_TOP,), jnp.int32),
            pltpu.VMEM((_PC,), jnp.int32),
            pltpu.VMEM((_PC,), jnp.int32),
            pltpu.VMEM((_PCX, _D), jnp.float32),
            pltpu.VMEM((_PCX, _D), jnp.float32),
            pltpu.VMEM((_PC, _EMB), jnp.float32),
            pltpu.SemaphoreType.DMA,
            pltpu.SemaphoreType.DMA,
            pltpu.SemaphoreType.DMA,
        ],
    )
    def k(ranks_h, xraw_h, xind_h, xflat_h, fc1_h,
          xg_h, sel_h, rg_h, tixs_h, tgis_h,
          rank_c, val_c, gid_c, tix, trg, tgi, idx_c, gidx_c,
          xrow_a, xrow_b, selb, sem_a, sem_b, sem_c):
        core = lax.axis_index("c")
        s = lax.axis_index("s")

        @pl.when(s < 1)
        def phase1():
            b = b0 + core
            base = b * _N

            def outer(cc, carry):
                off = base + cc * _CH
                pltpu.sync_copy(ranks_h.at[pl.ds(off, _CH)], rank_c)
                pltpu.sync_copy(xraw_h.at[pl.ds(off, _CH)], val_c)
                pltpu.sync_copy(xind_h.at[pl.ds(off, _CH)], gid_c)

                def inner(kk, carry2):
                    rv = rank_c[pl.ds(kk * 16, 16)]
                    vv = val_c[pl.ds(kk * 16, 16)]
                    gv = gid_c[pl.ds(kk * 16, 16)]
                    ig = (off + kk * 16
                          + lax.broadcasted_iota(jnp.int32, (16,), 0))
                    m = rv < _TOP
                    plsc.store_scatter(tix, [rv], ig, mask=m)
                    plsc.store_scatter(trg, [rv], vv, mask=m)
                    plsc.store_scatter(tgi, [rv], gv, mask=m)
                    return carry2

                lax.fori_loop(0, _CH // 16, inner, 0)
                return carry

            lax.fori_loop(0, _N // _CH, outer, 0)

            stage = core * _TOP
            pltpu.sync_copy(tix, tixs_h.at[pl.ds(stage, _TOP)])
            pltpu.sync_copy(tgi, tgis_h.at[pl.ds(stage, _TOP)])
            pltpu.sync_copy(trg, rg_h.at[pl.ds(stage, _TOP)])

        plsc.subcore_barrier()

        for q in range(2):
            c = q * 16 + (15 - s)

            @pl.when(c < nchunks)
            def chunk():
                cbase = core * _TOP + c * _PC
                pltpu.sync_copy(tixs_h.at[pl.ds(cbase, _PC)], idx_c)
                pltpu.sync_copy(tgis_h.at[pl.ds(cbase, _PC)], gidx_c)
                selcp = pltpu.async_copy(fc1_h.at[gidx_c], selb, sem_c)
                nx = _PC // _PCX
                xbufs = (xrow_a, xrow_b)
                xsems = (sem_a, sem_b)
                cps = [None, None]
                cps[0] = pltpu.async_copy(
                    xflat_h.at[idx_c.at[pl.ds(0, _PCX)]], xrow_a, sem_a)
                for i in range(nx):
                    bsl = i % 2
                    if i + 1 < nx:
                        cps[(i + 1) % 2] = pltpu.async_copy(
                            xflat_h.at[idx_c.at[pl.ds((i + 1) * _PCX, _PCX)]],
                            xbufs[(i + 1) % 2], xsems[(i + 1) % 2])
                    cps[bsl].wait()
                    pltpu.sync_copy(xbufs[bsl],
                                    xg_h.at[pl.ds(cbase + i * _PCX, _PCX)])
                selcp.wait()
                pltpu.sync_copy(selb, sel_h.at[pl.ds(cbase, _PC)])

    return k(ranks, xraw, xind, xflat, fc1)


_RB = 1000


def _mlp_body(xg_ref, sel_ref, rg_ref, wx_ref, ws_ref, wr_ref, cb_ref,
              t1w_ref, t1b_ref, t2w_ref, t2b_ref, lng_ref, lnb_ref,
              pw_ref, pb_ref, o_ref):
    r = rg_ref[...]
    h1 = jnp.maximum(r * t1w_ref[...] + t1b_ref[...], 0.0)
    remb = (jnp.dot(h1.astype(jnp.bfloat16),
                    t2w_ref[...].astype(jnp.bfloat16),
                    preferred_element_type=jnp.float32)
            + t2b_ref[...])
    h2 = (jnp.dot(xg_ref[...].astype(jnp.bfloat16),
                  wx_ref[...].astype(jnp.bfloat16),
                  preferred_element_type=jnp.float32)
          + jnp.dot(sel_ref[...].astype(jnp.bfloat16),
                    ws_ref[...].astype(jnp.bfloat16),
                    preferred_element_type=jnp.float32)
          + jnp.dot(remb.astype(jnp.bfloat16),
                    wr_ref[...].astype(jnp.bfloat16),
                    preferred_element_type=jnp.float32)
          + cb_ref[...])
    mu = jnp.mean(h2, axis=1, keepdims=True)
    d0 = h2 - mu
    var = jnp.mean(d0 * d0, axis=1, keepdims=True)
    hn = d0 * lax.rsqrt(var + 1e-5) * lng_ref[...] + lnb_ref[...]
    hg = hn * (1.0 / (1.0 + jnp.exp(-1.702 * hn)))
    o_ref[...] = (jnp.dot(hg.astype(jnp.bfloat16),
                          pw_ref[...].astype(jnp.bfloat16),
                          preferred_element_type=jnp.float32)
                  + pb_ref[...])


def _mlp_call(xg, sel, rg2d, wx, ws, wr, cb, t1w, t1b, t2w, t2b,
              lng, lnb, pw, pb):
    full = lambda shape: pl.BlockSpec(shape, lambda i: tuple(0 for _ in shape))
    return pl.pallas_call(
        _mlp_body,
        grid=(2 * _TOP // _RB,),
        in_specs=[
            pl.BlockSpec((_RB, _D), lambda i: (i, 0)),
            pl.BlockSpec((_RB, _EMB), lambda i: (i, 0)),
            pl.BlockSpec((_RB, 1), lambda i: (i, 0)),
            full((_D, _D)),
            full((_EMB, _D)),
            full((_EMB, _D)),
            full((1, _D)),
            full((1, 50)),
            full((1, 50)),
            full((50, _EMB)),
            full((1, _EMB)),
            full((1, _D)),
            full((1, _D)),
            full((_D, _D)),
            full((1, _D)),
        ],
        out_specs=pl.BlockSpec((_RB, _D), lambda i: (i, 0)),
        out_shape=jax.ShapeDtypeStruct((2 * _TOP, _D), jnp.float32),
    )(xg, sel, rg2d, wx, ws, wr, cb, t1w, t1b, t2w, t2b, lng, lnb, pw, pb)


def kernel(x, x_raw, x_indices, grn_emb, ppi_emb, fc1_w, fc1_b, t1_w, t1_b,
           t2_w, t2_b, cat_fc_w, cat_fc_b, ln_g, ln_b, proj_w, proj_b):
    ranks, fc1_out = _rank_fc1_call(x_raw, grn_emb, ppi_emb, fc1_w,
                                    fc1_b.reshape(1, _EMB))
    rflat = ranks.reshape(-1)
    xrflat = x_raw.reshape(-1)
    xiflat = x_indices.reshape(-1)
    xflat = x.reshape(_B * _N, _D)
    mlp_w = (cat_fc_w[:_D], cat_fc_w[_D:_D + _EMB], cat_fc_w[_D + _EMB:],
             cat_fc_b.reshape(1, _D), t1_w, t1_b.reshape(1, 50), t2_w,
             t2_b.reshape(1, _EMB), ln_g.reshape(1, _D), ln_b.reshape(1, _D),
             proj_w, proj_b.reshape(1, _D))
    halves = []
    for b0 in (0, 2):
        xg, sel, rg, _, _ = _sc_select_gather(
            b0, rflat, xrflat, xiflat, xflat, fc1_out)
        halves.append((xg, sel, rg))
    ys = [_mlp_call(xg, sel, rg.reshape(2 * _TOP, 1), *mlp_w)
          for (xg, sel, rg) in halves]
    return jnp.concatenate(ys, axis=0).reshape(_B, _TOP, _D)


# speedup vs baseline: 1.1646x; 1.1646x over previous
import functools

import jax
import jax.numpy as jnp
from jax import lax
from jax.experimental import pallas as pl
from jax.experimental.pallas import tpu as pltpu
from jax.experimental.pallas import tpu_sc as plsc

_G = 17911
_B, _N, _D = 4, 8192, 768
_EMB = 128
_TOP = 5000
_RTOT = _B * _TOP

def _mono(v):
    b = lax.bitcast_convert_type(v, jnp.int32)
    return jnp.where(b >= 0, b, jnp.full_like(b, 2147483647) - b)


_KB = 1024
_NB = _N // _KB
_GBF = 4480


def _rank_body(xrow_ref, grn_ref, ppi_ref, w_ref, b_ref, o_ref, f_ref,
               mrow, macc):
    w = w_ref[...]
    f_ref[...] = (
        jnp.dot(grn_ref[...], w[:_EMB, :], preferred_element_type=jnp.float32)
        + jnp.dot(ppi_ref[...], w[_EMB:, :], preferred_element_type=jnp.float32)
        + b_ref[...]
    )

    mrow[0, :] = _mono(xrow_ref[0, 0, :])
    ii = lax.broadcasted_iota(jnp.int32, (_KB, _KB), 1)
    jj = lax.broadcasted_iota(jnp.int32, (_KB, _KB), 0)
    tie_mask = jj < ii

    def a_loop(a, carry):
        mi = mrow[0, pl.ds(a * _KB, _KB)][None, :]
        macc[...] = ((mrow[0, pl.ds(a * _KB, _KB)][:, None] > mi)
                     | ((mrow[0, pl.ds(a * _KB, _KB)][:, None] == mi)
                        & tie_mask)).astype(jnp.float32)

        def lo_loop(b, carry2):
            mj = mrow[0, pl.ds(b * _KB, _KB)][:, None]
            macc[...] += (mj >= mi).astype(jnp.float32)
            return carry2

        def hi_loop(b, carry2):
            mj = mrow[0, pl.ds(b * _KB, _KB)][:, None]
            macc[...] += (mj > mi).astype(jnp.float32)
            return carry2

        lax.fori_loop(0, a, lo_loop, 0)
        lax.fori_loop(a + 1, _NB, hi_loop, 0)
        o_ref[0, 0, pl.ds(a * _KB, _KB)] = jnp.sum(
            macc[...], axis=0).astype(jnp.int32)
        return carry

    lax.fori_loop(0, _NB, a_loop, 0)


def _rank_fc1_call(x_raw, grn, ppi, w, b2d):
    x3 = x_raw.reshape(_B, 1, _N)
    ranks, fc1 = pl.pallas_call(
        _rank_body,
        grid=(_B,),
        in_specs=[
            pl.BlockSpec((1, 1, _N), lambda b: (b, 0, 0)),
            pl.BlockSpec((_GBF, _EMB), lambda b: (b, 0)),
            pl.BlockSpec((_GBF, _EMB), lambda b: (b, 0)),
            pl.BlockSpec((2 * _EMB, _EMB), lambda b: (0, 0)),
            pl.BlockSpec((1, _EMB), lambda b: (0, 0)),
        ],
        out_specs=[
            pl.BlockSpec((1, 1, _N), lambda b: (b, 0, 0)),
            pl.BlockSpec((_GBF, _EMB), lambda b: (b, 0)),
        ],
        out_shape=[
            jax.ShapeDtypeStruct((_B, 1, _N), jnp.int32),
            jax.ShapeDtypeStruct((_G, _EMB), jnp.float32),
        ],
        scratch_shapes=[
            pltpu.VMEM((1, _N), jnp.int32),
            pltpu.VMEM((_KB, _KB), jnp.float32),
        ],
    )(x3, grn, ppi, w, b2d)
    return ranks.reshape(_B, _N), fc1


_CH = 2048
_PC = 200
_PCX = 40


def _sc_select_gather(ranks, xraw, xind, xflat, fc1):
    mesh = plsc.VectorSubcoreMesh(core_axis_name="c", subcore_axis_name="s")
    core_rows = 2 * _TOP
    nchunks = core_rows // _PC

    @functools.partial(
        pl.kernel,
        mesh=mesh,
        compiler_params=pltpu.CompilerParams(needs_layout_passes=False),
        out_type=[
            jax.ShapeDtypeStruct((_RTOT, _D), jnp.float32),
            jax.ShapeDtypeStruct((_RTOT, _EMB), jnp.float32),
            jax.ShapeDtypeStruct((_RTOT,), jnp.float32),
            jax.ShapeDtypeStruct((_RTOT,), jnp.int32),
            jax.ShapeDtypeStruct((_RTOT,), jnp.int32),
        ],
        scratch_types=[
            pltpu.VMEM((_CH,), jnp.int32),
            pltpu.VMEM((_CH,), jnp.float32),
            pltpu.VMEM((_CH,), jnp.int32),
            pltpu.VMEM((_TOP,), jnp.int32),
            pltpu.VMEM((_TOP,), jnp.float32),
            pltpu.VMEM((_TOP,), jnp.int32),
            pltpu.VMEM((_PC,), jnp.int32),
            pltpu.VMEM((_PC,), jnp.int32),
            pltpu.VMEM((_PCX, _D), jnp.float32),
            pltpu.VMEM((_PCX, _D), jnp.float32),
            pltpu.VMEM((_PC, _EMB), jnp.float32),
            pltpu.SemaphoreType.DMA,
            pltpu.SemaphoreType.DMA,
            pltpu.SemaphoreType.DMA,
        ],
    )
    def k(ranks_h, xraw_h, xind_h, xflat_h, fc1_h,
          xg_h, sel_h, rg_h, tixs_h, tgis_h,
          rank_c, val_c, gid_c, tix, trg, tgi, idx_c, gidx_c,
          xrow_a, xrow_b, selb, sem_a, sem_b, sem_c):
        core = lax.axis_index("c")
        s = lax.axis_index("s")

        @pl.when(s < 2)
        def phase1():
            b = core * 2 + s
            base = b * _N

            def outer(cc, carry):
                off = base + cc * _CH
                pltpu.sync_copy(ranks_h.at[pl.ds(off, _CH)], rank_c)
                pltpu.sync_copy(xraw_h.at[pl.ds(off, _CH)], val_c)
                pltpu.sync_copy(xind_h.at[pl.ds(off, _CH)], gid_c)

                def inner(kk, carry2):
                    rv = rank_c[pl.ds(kk * 16, 16)]
                    vv = val_c[pl.ds(kk * 16, 16)]
                    gv = gid_c[pl.ds(kk * 16, 16)]
                    ig = (off + kk * 16
                          + lax.broadcasted_iota(jnp.int32, (16,), 0))
                    m = rv < _TOP
                    plsc.store_scatter(tix, [rv], ig, mask=m)
                    plsc.store_scatter(trg, [rv], vv, mask=m)
                    plsc.store_scatter(tgi, [rv], gv, mask=m)
                    return carry2

                lax.fori_loop(0, _CH // 16, inner, 0)
                return carry

            lax.fori_loop(0, _N // _CH, outer, 0)

            stage = core * core_rows + s * _TOP
            pltpu.sync_copy(tix, tixs_h.at[pl.ds(stage, _TOP)])
            pltpu.sync_copy(tgi, tgis_h.at[pl.ds(stage, _TOP)])
            pltpu.sync_copy(trg, rg_h.at[pl.ds(b * _TOP, _TOP)])

        plsc.subcore_barrier()

        for q in range(4):
            c = q * 16 + (15 - s)

            @pl.when(c < nchunks)
            def chunk():
                cbase = core * core_rows + c * _PC
                pltpu.sync_copy(tixs_h.at[pl.ds(cbase, _PC)], idx_c)
                pltpu.sync_copy(tgis_h.at[pl.ds(cbase, _PC)], gidx_c)
                selcp = pltpu.async_copy(fc1_h.at[gidx_c], selb, sem_c)
                nx = _PC // _PCX
                xbufs = (xrow_a, xrow_b)
                xsems = (sem_a, sem_b)
                cps = [None, None]
                cps[0] = pltpu.async_copy(
                    xflat_h.at[idx_c.at[pl.ds(0, _PCX)]], xrow_a, sem_a)
                for i in range(nx):
                    bsl = i % 2
                    if i + 1 < nx:
                        cps[(i + 1) % 2] = pltpu.async_copy(
                            xflat_h.at[idx_c.at[pl.ds((i + 1) * _PCX, _PCX)]],
                            xbufs[(i + 1) % 2], xsems[(i + 1) % 2])
                    cps[bsl].wait()
                    pltpu.sync_copy(xbufs[bsl],
                                    xg_h.at[pl.ds(cbase + i * _PCX, _PCX)])
                selcp.wait()
                pltpu.sync_copy(selb, sel_h.at[pl.ds(cbase, _PC)])

    return k(ranks, xraw, xind, xflat, fc1)


_RB = 800


def _mlp_body(xg_ref, sel_ref, rg_ref, wx_ref, ws_ref, wr_ref, cb_ref,
              t1w_ref, t1b_ref, t2w_ref, t2b_ref, lng_ref, lnb_ref,
              pw_ref, pb_ref, o_ref):
    r = rg_ref[...]
    h1 = jnp.maximum(r * t1w_ref[...] + t1b_ref[...], 0.0)
    remb = (jnp.dot(h1.astype(jnp.bfloat16),
                    t2w_ref[...].astype(jnp.bfloat16),
                    preferred_element_type=jnp.float32)
            + t2b_ref[...])
    h2 = (jnp.dot(xg_ref[...].astype(jnp.bfloat16),
                  wx_ref[...].astype(jnp.bfloat16),
                  preferred_element_type=jnp.float32)
          + jnp.dot(sel_ref[...].astype(jnp.bfloat16),
                    ws_ref[...].astype(jnp.bfloat16),
                    preferred_element_type=jnp.float32)
          + jnp.dot(remb.astype(jnp.bfloat16),
                    wr_ref[...].astype(jnp.bfloat16),
                    preferred_element_type=jnp.float32)
          + cb_ref[...])
    mu = jnp.mean(h2, axis=1, keepdims=True)
    d0 = h2 - mu
    var = jnp.mean(d0 * d0, axis=1, keepdims=True)
    hn = d0 * lax.rsqrt(var + 1e-5) * lng_ref[...] + lnb_ref[...]
    hg = hn * (1.0 / (1.0 + jnp.exp(-1.702 * hn)))
    o_ref[...] = (jnp.dot(hg.astype(jnp.bfloat16),
                          pw_ref[...].astype(jnp.bfloat16),
                          preferred_element_type=jnp.float32)
                  + pb_ref[...])


def _mlp_call(xg, sel, rg2d, wx, ws, wr, cb, t1w, t1b, t2w, t2b,
              lng, lnb, pw, pb):
    full = lambda shape: pl.BlockSpec(shape, lambda i: tuple(0 for _ in shape))
    return pl.pallas_call(
        _mlp_body,
        grid=(_RTOT // _RB,),
        in_specs=[
            pl.BlockSpec((_RB, _D), lambda i: (i, 0)),
            pl.BlockSpec((_RB, _EMB), lambda i: (i, 0)),
            pl.BlockSpec((_RB, 1), lambda i: (i, 0)),
            full((_D, _D)),
            full((_EMB, _D)),
            full((_EMB, _D)),
            full((1, _D)),
            full((1, 50)),
            full((1, 50)),
            full((50, _EMB)),
            full((1, _EMB)),
            full((1, _D)),
            full((1, _D)),
            full((_D, _D)),
            full((1, _D)),
        ],
        out_specs=pl.BlockSpec((_RB, _D), lambda i: (i, 0)),
        out_shape=jax.ShapeDtypeStruct((_RTOT, _D), jnp.float32),
    )(xg, sel, rg2d, wx, ws, wr, cb, t1w, t1b, t2w, t2b, lng, lnb, pw, pb)


def kernel(x, x_raw, x_indices, grn_emb, ppi_emb, fc1_w, fc1_b, t1_w, t1_b,
           t2_w, t2_b, cat_fc_w, cat_fc_b, ln_g, ln_b, proj_w, proj_b):
    ranks, fc1_out = _rank_fc1_call(x_raw, grn_emb, ppi_emb, fc1_w,
                                    fc1_b.reshape(1, _EMB))
    xg, sel, rg, _, _ = _sc_select_gather(
        ranks.reshape(-1),
        x_raw.reshape(-1),
        x_indices.reshape(-1),
        x.reshape(_B * _N, _D),
        fc1_out,
    )
    y = _mlp_call(
        xg, sel, rg.reshape(_RTOT, 1),
        cat_fc_w[:_D], cat_fc_w[_D:_D + _EMB], cat_fc_w[_D + _EMB:],
        cat_fc_b.reshape(1, _D), t1_w, t1_b.reshape(1, 50), t2_w,
        t2_b.reshape(1, _EMB), ln_g.reshape(1, _D), ln_b.reshape(1, _D),
        proj_w, proj_b.reshape(1, _D),
    )
    return y.reshape(_B, _TOP, _D)
